# Initial kernel scaffold; baseline (speedup 1.0000x reference)
import jax
import jax.numpy as jnp
from jax.experimental import pallas as pl


def kernel(inp, regular_tokens_mask, token_ids):
    raise NotImplementedError("write your pallas kernel here")


# speedup vs baseline: 89.4164x; 89.4164x over previous
import jax
import jax.numpy as jnp
from jax.experimental import pallas as pl

_B, _L = 8, 2048
_EXT_MAX_ID = 20030


def _body(mask_ref, tid_ref, out_ref):
    m = (mask_ref[...] == 1).astype(jnp.int32)
    tid = tid_ref[...]
    small = (tid <= _EXT_MAX_ID).astype(jnp.int32)
    ext = m[:, :-1] * m[:, 1:] * small[:, :-1]
    a = jnp.concatenate([jnp.zeros((_B, 1), jnp.int32), ext], axis=1)
    idx = jax.lax.broadcasted_iota(jnp.int32, (_B, _L), 1)
    v = jnp.where(a == 1, -1, idx)
    d = 1
    while d < _L:
        shifted = jnp.concatenate(
            [jnp.full((_B, d), -1, jnp.int32), v[:, : _L - d]], axis=1)
        v = jnp.maximum(v, shifted)
        d *= 2
    n = 1 - ((idx - v) & 1)
    d = 1
    while d < _L:
        shifted = jnp.concatenate(
            [jnp.zeros((_B, d), jnp.int32), n[:, : _L - d]], axis=1)
        n = n + shifted
        d *= 2
    out_ref[...] = n - 1


def kernel(inp, regular_tokens_mask, token_ids):
    del inp
    return pl.pallas_call(
        _body,
        out_shape=jax.ShapeDtypeStruct((_B, _L), jnp.int32),
    )(regular_tokens_mask, token_ids)
